# baseline (speedup 1.0000x reference)
import functools

import jax
import jax.numpy as jnp
from jax import lax
from jax.experimental import pallas as pl
from jax.experimental.pallas import tpu as pltpu
from jax.experimental.pallas import tpu_sc as plsc

_N = 10000
_E = 160000
_EPS = 1e-5
_NC = 2
_NS = 16
_NH = _N // _NC
_B = 125
_NBA = _E // _NS // _B
_TRASH = _NH
_AROWS = 5008
_ARPW = _AROWS // _NS
_DW = 128
_R = 200


@functools.cache
def _sc_kernels():
    mesh = plsc.VectorSubcoreMesh(core_axis_name="c", subcore_axis_name="s")

    @functools.partial(
        pl.kernel,
        out_type=jax.ShapeDtypeStruct((_NC, _NS, _ARPW, _DW), jnp.float32),
        mesh=mesh,
        scratch_types=[
            pltpu.VMEM((_NBA, _B), jnp.int32),
            pltpu.VMEM((_B, _DW), jnp.float32),
            pltpu.VMEM_SHARED((_AROWS, _DW), jnp.float32),
            pltpu.SemaphoreType.DMA,
        ],
    )
    def deg(dstt_hbm, out_hbm, dstv, ones_v, acc, ssem):
        cid = lax.axis_index("c")
        sid = lax.axis_index("s")

        def fill(val):
            def go(r, carry):
                for k in range(_DW // 16):
                    ones_v[r, pl.ds(k * 16, 16)] = jnp.full((16,), val,
                                                            jnp.float32)
                return carry
            return go

        lax.fori_loop(0, _B, fill(0.0), 0)
        base = sid * _ARPW
        for i in range(2):
            pltpu.sync_copy(ones_v, acc.at[pl.ds(base + i * _B, _B)])
        pltpu.sync_copy(ones_v.at[pl.ds(0, _ARPW - 2 * _B)],
                        acc.at[pl.ds(base + 2 * _B, _ARPW - 2 * _B)])
        lax.fori_loop(0, _B, fill(1.0), 0)
        pltpu.sync_copy(dstt_hbm.at[cid, sid], dstv)
        plsc.subcore_barrier()

        def body(t, carry):
            pltpu.sync_copy(ones_v, acc.at[dstv.at[t]], add=True)
            return carry

        lax.fori_loop(0, _NBA, body, 0)
        plsc.subcore_barrier()
        pltpu.sync_copy(acc.at[pl.ds(base, _ARPW)], out_hbm.at[cid, sid])

    @functools.partial(
        pl.kernel,
        out_type=jax.ShapeDtypeStruct((_NC, 2, _NS, _ARPW, 128),
                                      jnp.float32),
        mesh=mesh,
        scratch_types=[
            pltpu.VMEM((_NBA, _B), jnp.int32),
            pltpu.VMEM((_NBA, _B), jnp.int32),
            pltpu.VMEM((_B, 128), jnp.float32),
            pltpu.VMEM((_B, 128), jnp.float32),
            pltpu.VMEM((_B, 128), jnp.float32),
            pltpu.VMEM((_B, 128), jnp.float32),
            pltpu.VMEM_SHARED((_AROWS, 128), jnp.float32),
            pltpu.SemaphoreType.DMA,
            pltpu.SemaphoreType.DMA,
        ],
    )
    def agg1(tab_hbm, src_hbm, dstt_hbm, out_hbm, srcv, dstv, r0, r1, r2,
             r3, acc, gsem, ssem):
        cid = lax.axis_index("c")
        sid = lax.axis_index("s")
        rows = [r0, r1, r2, r3]

        def fillz(r, carry):
            for k in range(8):
                r0[r, pl.ds(k * 16, 16)] = jnp.zeros((16,), jnp.float32)
            return carry

        base = sid * _ARPW
        pltpu.sync_copy(src_hbm.at[cid, sid], srcv)
        for p in range(2):
            lax.fori_loop(0, _B, fillz, 0)
            for i in range(2):
                pltpu.sync_copy(r0, acc.at[pl.ds(base + i * _B, _B)])
            pltpu.sync_copy(r0.at[pl.ds(0, _ARPW - 2 * _B)],
                            acc.at[pl.ds(base + 2 * _B, _ARPW - 2 * _B)])
            pltpu.sync_copy(dstt_hbm.at[p, sid], dstv)
            plsc.subcore_barrier()

            def body(t, carry):
                j0 = t * 4
                gd = [pltpu.async_copy(tab_hbm.at[srcv.at[j0 + u]],
                                       rows[u], gsem) for u in range(4)]
                for dsc in gd:
                    dsc.wait()
                sd = [pltpu.async_copy(rows[u], acc.at[dstv.at[j0 + u]],
                                       ssem, add=True) for u in range(4)]
                for dsc in sd:
                    dsc.wait()
                return carry

            lax.fori_loop(0, _NBA // 4, body, 0)
            plsc.subcore_barrier()
            pltpu.sync_copy(acc.at[pl.ds(base, _ARPW)],
                            out_hbm.at[cid, p, sid])

    @functools.partial(
        pl.kernel,
        out_type=jax.ShapeDtypeStruct((_NC, _NS, _ARPW, 128), jnp.float32),
        mesh=mesh,
        scratch_types=[
            pltpu.VMEM((_NBA, _B), jnp.int32),
            pltpu.VMEM((_NBA, _B), jnp.int32),
            pltpu.VMEM((_B, 128), jnp.float32),
            pltpu.VMEM((_B, 128), jnp.float32),
            pltpu.VMEM((_B, 128), jnp.float32),
            pltpu.VMEM((_B, 128), jnp.float32),
            pltpu.VMEM_SHARED((_AROWS, 128), jnp.float32),
            pltpu.SemaphoreType.DMA,
            pltpu.SemaphoreType.DMA,
        ],
    )
    def agg2(tab_hbm, src_hbm, dstt_hbm, out_hbm, srcv, dstv, r0, r1, r2,
             r3, acc, gsem, ssem):
        cid = lax.axis_index("c")
        sid = lax.axis_index("s")
        rows = [r0, r1, r2, r3]

        def fillz(r, carry):
            for k in range(8):
                r0[r, pl.ds(k * 16, 16)] = jnp.zeros((16,), jnp.float32)
            return carry

        lax.fori_loop(0, _B, fillz, 0)
        base = sid * _ARPW
        for i in range(2):
            pltpu.sync_copy(r0, acc.at[pl.ds(base + i * _B, _B)])
        pltpu.sync_copy(r0.at[pl.ds(0, _ARPW - 2 * _B)],
                        acc.at[pl.ds(base + 2 * _B, _ARPW - 2 * _B)])
        pltpu.sync_copy(src_hbm.at[sid], srcv)
        pltpu.sync_copy(dstt_hbm.at[cid, sid], dstv)
        plsc.subcore_barrier()

        def body(t, carry):
            j0 = t * 4
            gd = [pltpu.async_copy(tab_hbm.at[srcv.at[j0 + u]],
                                   rows[u], gsem) for u in range(4)]
            for dsc in gd:
                dsc.wait()
            sd = [pltpu.async_copy(rows[u], acc.at[dstv.at[j0 + u]],
                                   ssem, add=True) for u in range(4)]
            for dsc in sd:
                dsc.wait()
            return carry

        lax.fori_loop(0, _NBA // 4, body, 0)
        plsc.subcore_barrier()
        pltpu.sync_copy(acc.at[pl.ds(base, _ARPW)], out_hbm.at[cid, sid])

    return deg, agg1, agg2


def _dot(a, b):
    return lax.dot_general(a, b, (((1,), (0,)), ((), ())),
                           preferred_element_type=jnp.float32,
                           precision=lax.Precision.HIGHEST)


def _dinv_of(degp):
    return lax.rsqrt(degp[0, :, 0:1] + 1.0)


def _xf_body(dst_ref, out_ref):
    v = dst_ref[...]
    out_ref[0] = jnp.where(v < _NH, v, _TRASH)
    v1 = v - _NH
    out_ref[1] = jnp.where(v1 >= 0, v1, _TRASH)


def _tc1_body(x_ref, lng, lnb, w1, degp, out_ref):
    xb = x_ref[...]
    mu = jnp.mean(xb, axis=1, keepdims=True)
    xc = xb - mu
    var = jnp.mean(xc * xc, axis=1, keepdims=True)
    h = xc * lax.rsqrt(var + _EPS) * lng[0] + lnb[0]
    hw = _dot(h, w1[...]) * _dinv_of(degp)
    out_ref[0] = hw[:, 0:128]
    out_ref[1] = hw[:, 128:256]


def _tc2_body(agg, tabp, degp, b1, g1, be1, m1, v1, w2, out_ref):
    dinv = _dinv_of(degp)
    h = jnp.concatenate([agg[0, 0] + tabp[0], agg[1, 0] + tabp[1]], axis=1)
    h = h * dinv + b1[0]
    h = (h - m1[0]) * lax.rsqrt(v1[0] + _EPS) * g1[0] + be1[0]
    h = jnp.maximum(h, 0.0)
    out_ref[...] = _dot(h, w2[...]) * dinv


def _tc3_body(agg, tabp, degp, b2, g2, be2, m2, v2, wc1, bc1, lncg, lncb,
              wc2, bc2, out_ref):
    dinv = _dinv_of(degp)
    h = agg[0] + tabp[...]
    h = h * dinv + b2[0]
    h = (h - m2[0]) * lax.rsqrt(v2[0] + _EPS) * g2[0] + be2[0]
    h = jnp.maximum(h, 0.0)
    hc = _dot(h, wc1[...]) + bc1[0]
    mu = jnp.mean(hc, axis=1, keepdims=True)
    hcc = hc - mu
    var = jnp.mean(hcc * hcc, axis=1, keepdims=True)
    hc = hcc * lax.rsqrt(var + _EPS) * lncg[0] + lncb[0]
    hc = jnp.maximum(hc, 0.0)
    out_ref[...] = _dot(hc, wc2[...]) + bc2[0]


def _bcast_spec(shape):
    return pl.BlockSpec(shape, lambda i: tuple(0 for _ in shape))


_deg_spec = pl.BlockSpec((1, _R, _DW), lambda i: (i // (_NH // _R),
                                                  i % (_NH // _R), 0))

_xf = pl.pallas_call(
    _xf_body,
    grid=(1,),
    in_specs=[pl.BlockSpec((_E // 128, 128), lambda i: (0, 0))],
    out_specs=pl.BlockSpec((2, _E // 128, 128), lambda i: (0, 0, 0)),
    out_shape=jax.ShapeDtypeStruct((2, _E // 128, 128), jnp.int32),
)

_tc1 = pl.pallas_call(
    _tc1_body,
    grid=(_N // _R,),
    in_specs=[
        pl.BlockSpec((_R, 256), lambda i: (i, 0)),
        _bcast_spec((1, 256)),
        _bcast_spec((1, 256)),
        _bcast_spec((256, 256)),
        _deg_spec,
    ],
    out_specs=pl.BlockSpec((2, _R, 128), lambda i: (0, i, 0)),
    out_shape=jax.ShapeDtypeStruct((2, _N, 128), jnp.float32),
)

_tc2 = pl.pallas_call(
    _tc2_body,
    grid=(_N // _R,),
    in_specs=[
        pl.BlockSpec((2, 1, _R, 128), lambda i: (0, i // (_NH // _R),
                                                 i % (_NH // _R), 0)),
        pl.BlockSpec((2, _R, 128), lambda i: (0, i, 0)),
        _deg_spec,
        _bcast_spec((1, 256)),
        _bcast_spec((1, 256)),
        _bcast_spec((1, 256)),
        _bcast_spec((1, 256)),
        _bcast_spec((1, 256)),
        _bcast_spec((256, 128)),
    ],
    out_specs=pl.BlockSpec((_R, 128), lambda i: (i, 0)),
    out_shape=jax.ShapeDtypeStruct((_N, 128), jnp.float32),
)

_tc3 = pl.pallas_call(
    _tc3_body,
    grid=(_N // _R,),
    in_specs=[
        pl.BlockSpec((1, _R, 128), lambda i: (i // (_NH // _R),
                                              i % (_NH // _R), 0)),
        pl.BlockSpec((_R, 128), lambda i: (i, 0)),
        _deg_spec,
        _bcast_spec((1, 128)),
        _bcast_spec((1, 128)),
        _bcast_spec((1, 128)),
        _bcast_spec((1, 128)),
        _bcast_spec((1, 128)),
        _bcast_spec((128, 64)),
        _bcast_spec((1, 64)),
        _bcast_spec((1, 64)),
        _bcast_spec((1, 64)),
        _bcast_spec((64, 8)),
        _bcast_spec((1, 8)),
    ],
    out_specs=pl.BlockSpec((_R, 8), lambda i: (i, 0)),
    out_shape=jax.ShapeDtypeStruct((_N, 8), jnp.float32),
)


def kernel(x, edge_index, ln_g, ln_b, W1, b1, bn1_g, bn1_b, bn1_m, bn1_v,
           W2, b2, bn2_g, bn2_b, bn2_m, bn2_v, Wc1, bc1, lnc_g, lnc_b,
           Wc2, bc2):
    src = edge_index[0]
    dst = edge_index[1]
    src_rs = src.reshape(_NS, _NBA, _B)
    src_both = jnp.stack([src_rs, src_rs + _N])

    r2 = lambda a: a.reshape(1, -1)

    _deg, _agg1, _agg2 = _sc_kernels()
    dstt = _xf(dst.reshape(_E // 128, 128)).reshape(2, _NS, _NBA, _B)
    degp = _deg(dstt).reshape(_NC, _AROWS, _DW)
    tab1 = _tc1(x, r2(ln_g), r2(ln_b), W1, degp)
    agg1 = _agg1(tab1.reshape(2 * _N, 128), src_both, dstt)
    tab2 = _tc2(agg1.reshape(_NC, 2, _AROWS, 128), tab1, degp, r2(b1),
                r2(bn1_g), r2(bn1_b), r2(bn1_m), r2(bn1_v), W2)
    agg2 = _agg2(tab2, src_rs, dstt)
    out = _tc3(agg2.reshape(_NC, _AROWS, 128), tab2, degp, r2(b2),
               r2(bn2_g), r2(bn2_b), r2(bn2_m), r2(bn2_v), Wc1, r2(bc1),
               r2(lnc_g), r2(lnc_b), Wc2, r2(bc2))
    return out


# speedup vs baseline: 1.3645x; 1.3645x over previous
import functools

import jax
import jax.numpy as jnp
from jax import lax
from jax.experimental import pallas as pl
from jax.experimental.pallas import tpu as pltpu
from jax.experimental.pallas import tpu_sc as plsc

_N = 10000
_E = 160000
_EPS = 1e-5
_NC = 2
_NS = 16
_NH = _N // _NC
_B = 125
_RPW = _N // _NS
_NBA = _E // _NS // _B
_NP = 10240
_RPQ = _NP // _NS
_NBD = _E // (_NC * _NS) // _B
_B2 = 80
_NB2 = _E // _NS // _B2
_A2TRASH = _NH
_A2ROWS = 5008
_A2RPW = _A2ROWS // _NS
_R = 200


@functools.cache
def _sc_kernels():
    mesh = plsc.VectorSubcoreMesh(core_axis_name="c", subcore_axis_name="s")

    @functools.partial(
        pl.kernel,
        out_type=jax.ShapeDtypeStruct((_NC, _NP, 16), jnp.float32),
        mesh=mesh,
        scratch_types=[
            pltpu.VMEM((_NBD, _B), jnp.int32),
            pltpu.VMEM((_B, 16), jnp.float32),
            pltpu.VMEM((128, 16), jnp.float32),
            pltpu.VMEM_SHARED((_NP, 16), jnp.float32),
        ],
    )
    def deg(dst_hbm, out_hbm, dstv, ones_v, zeros_v, acc):
        cid = lax.axis_index("c")
        sid = lax.axis_index("s")

        def fill(r, carry):
            @pl.when(r < _B)
            def _():
                ones_v[r, :] = jnp.full((16,), 1.0, jnp.float32)

            zeros_v[r, :] = jnp.zeros((16,), jnp.float32)
            return carry

        lax.fori_loop(0, 128, fill, 0)
        for i in range(_RPQ // 128):
            pltpu.sync_copy(zeros_v, acc.at[pl.ds(sid * _RPQ + i * 128,
                                                  128)])
        pltpu.sync_copy(dst_hbm.at[cid, sid], dstv)
        plsc.subcore_barrier()

        def body(j, carry):
            pltpu.sync_copy(ones_v, acc.at[dstv.at[j]], add=True)
            return carry

        lax.fori_loop(0, _NBD, body, 0)
        plsc.subcore_barrier()
        pltpu.sync_copy(acc.at[pl.ds(sid * _RPQ, _RPQ)],
                        out_hbm.at[cid, pl.ds(sid * _RPQ, _RPQ)])

    @functools.partial(
        pl.kernel,
        out_type=jax.ShapeDtypeStruct((_NC, _NS, _RPW, 128), jnp.float32),
        mesh=mesh,
        scratch_types=[
            pltpu.VMEM((_NBA, _B), jnp.int32),
            pltpu.VMEM((_NBA, _B), jnp.int32),
            pltpu.VMEM((_B, 128), jnp.float32),
            pltpu.VMEM_SHARED((_N, 128), jnp.float32),
            pltpu.SemaphoreType.DMA,
        ],
    )
    def agg1(tab_hbm, src_hbm, dst_hbm, out_hbm, srcv, dstv, rows0, acc,
             sem0):
        cid = lax.axis_index("c")
        sid = lax.axis_index("s")

        def fillz(r, carry):
            for k in range(8):
                rows0[r, pl.ds(k * 16, 16)] = jnp.zeros((16,), jnp.float32)
            return carry

        lax.fori_loop(0, _B, fillz, 0)
        for i in range(_RPW // _B):
            pltpu.sync_copy(rows0, acc.at[pl.ds(sid * _RPW + i * _B, _B)])
        pltpu.sync_copy(src_hbm.at[cid, sid], srcv)
        pltpu.sync_copy(dst_hbm.at[sid], dstv)
        plsc.subcore_barrier()

        def body(j, carry):
            pltpu.async_copy(tab_hbm.at[srcv.at[j]], rows0, sem0).wait()
            pltpu.sync_copy(rows0, acc.at[dstv.at[j]], add=True)
            return carry

        lax.fori_loop(0, _NBA, body, 0)
        plsc.subcore_barrier()
        pltpu.sync_copy(acc.at[pl.ds(sid * _RPW, _RPW)],
                        out_hbm.at[cid, sid])

    @functools.partial(
        pl.kernel,
        out_type=jax.ShapeDtypeStruct((_NC, _NS, _A2RPW, 128), jnp.float32),
        mesh=mesh,
        scratch_types=[
            pltpu.VMEM((_NB2, _B2), jnp.int32),
            pltpu.VMEM((_NB2, _B2), jnp.int32),
            pltpu.VMEM((_NB2, _B2), jnp.int32),
            pltpu.VMEM((_B2, 128), jnp.float32),
            pltpu.VMEM_SHARED((_A2ROWS, 128), jnp.float32),
            pltpu.SemaphoreType.DMA,
        ],
    )
    def agg2(tab_hbm, src_hbm, dst_hbm, out_hbm, srcv, dstv, dstt, rows0,
             acc, sem0):
        cid = lax.axis_index("c")
        sid = lax.axis_index("s")
        lo = cid * _NH

        def fillz(r, carry):
            for k in range(8):
                rows0[r, pl.ds(k * 16, 16)] = jnp.zeros((16,), jnp.float32)
            return carry

        lax.fori_loop(0, _B2, fillz, 0)
        base = sid * _A2RPW
        for i in range(3):
            pltpu.sync_copy(rows0, acc.at[pl.ds(base + i * _B2, _B2)])
        pltpu.sync_copy(rows0.at[pl.ds(0, _A2RPW - 3 * _B2)],
                        acc.at[pl.ds(base + 3 * _B2, _A2RPW - 3 * _B2)])
        pltpu.sync_copy(src_hbm.at[sid], srcv)
        pltpu.sync_copy(dst_hbm.at[sid], dstv)

        def xform(j, carry):
            for k in range(_B2 // 16):
                v = dstv[j, pl.ds(k * 16, 16)]
                vl = v - lo
                ok = (vl >= 0) & (vl < _NH)
                dstt[j, pl.ds(k * 16, 16)] = jnp.where(ok, vl, _A2TRASH)
            return carry

        lax.fori_loop(0, _NB2, xform, 0)
        plsc.subcore_barrier()

        def body(j, carry):
            pltpu.async_copy(tab_hbm.at[srcv.at[j]], rows0, sem0).wait()
            pltpu.sync_copy(rows0, acc.at[dstt.at[j]], add=True)
            return carry

        lax.fori_loop(0, _NB2, body, 0)
        plsc.subcore_barrier()
        pltpu.sync_copy(acc.at[pl.ds(base, _A2RPW)], out_hbm.at[cid, sid])

    return deg, agg1, agg2


def _dot(a, b):
    return lax.dot_general(a, b, (((1,), (0,)), ((), ())),
                           preferred_element_type=jnp.float32,
                           precision=lax.Precision.HIGHEST)


def _dinv_of(degp):
    deg = degp[0, :, 0:1] + degp[1, :, 0:1] + 1.0
    return lax.rsqrt(deg)


def _tc1_body(x_ref, lng, lnb, w1, degp, out_ref):
    xb = x_ref[...]
    mu = jnp.mean(xb, axis=1, keepdims=True)
    xc = xb - mu
    var = jnp.mean(xc * xc, axis=1, keepdims=True)
    h = xc * lax.rsqrt(var + _EPS) * lng[0] + lnb[0]
    hw = _dot(h, w1[...]) * _dinv_of(degp)
    out_ref[0] = hw[:, 0:128]
    out_ref[1] = hw[:, 128:256]


def _tc2_body(agg, tabp, degp, b1, g1, be1, m1, v1, w2, out_ref):
    dinv = _dinv_of(degp)
    h = jnp.concatenate([agg[0] + tabp[0], agg[1] + tabp[1]], axis=1)
    h = h * dinv + b1[0]
    h = (h - m1[0]) * lax.rsqrt(v1[0] + _EPS) * g1[0] + be1[0]
    h = jnp.maximum(h, 0.0)
    out_ref[...] = _dot(h, w2[...]) * dinv


def _tc3_body(agg, tabp, degp, b2, g2, be2, m2, v2, wc1, bc1, lncg, lncb,
              wc2, bc2, out_ref):
    dinv = _dinv_of(degp)
    h = agg[0] + tabp[...]
    h = h * dinv + b2[0]
    h = (h - m2[0]) * lax.rsqrt(v2[0] + _EPS) * g2[0] + be2[0]
    h = jnp.maximum(h, 0.0)
    hc = _dot(h, wc1[...]) + bc1[0]
    mu = jnp.mean(hc, axis=1, keepdims=True)
    hcc = hc - mu
    var = jnp.mean(hcc * hcc, axis=1, keepdims=True)
    hc = hcc * lax.rsqrt(var + _EPS) * lncg[0] + lncb[0]
    hc = jnp.maximum(hc, 0.0)
    out_ref[...] = _dot(hc, wc2[...]) + bc2[0]


def _bcast_spec(shape):
    return pl.BlockSpec(shape, lambda i: tuple(0 for _ in shape))


_deg_spec = pl.BlockSpec((2, _R, 16), lambda i: (0, i, 0))

_tc1 = pl.pallas_call(
    _tc1_body,
    grid=(_N // _R,),
    in_specs=[
        pl.BlockSpec((_R, 256), lambda i: (i, 0)),
        _bcast_spec((1, 256)),
        _bcast_spec((1, 256)),
        _bcast_spec((256, 256)),
        _deg_spec,
    ],
    out_specs=pl.BlockSpec((2, _R, 128), lambda i: (0, i, 0)),
    out_shape=jax.ShapeDtypeStruct((2, _N, 128), jnp.float32),
)

_tc2 = pl.pallas_call(
    _tc2_body,
    grid=(_N // _R,),
    in_specs=[
        pl.BlockSpec((2, _R, 128), lambda i: (0, i, 0)),
        pl.BlockSpec((2, _R, 128), lambda i: (0, i, 0)),
        _deg_spec,
        _bcast_spec((1, 256)),
        _bcast_spec((1, 256)),
        _bcast_spec((1, 256)),
        _bcast_spec((1, 256)),
        _bcast_spec((1, 256)),
        _bcast_spec((256, 128)),
    ],
    out_specs=pl.BlockSpec((_R, 128), lambda i: (i, 0)),
    out_shape=jax.ShapeDtypeStruct((_N, 128), jnp.float32),
)

_tc3 = pl.pallas_call(
    _tc3_body,
    grid=(_N // _R,),
    in_specs=[
        pl.BlockSpec((1, _R, 128), lambda i: (i // (_NH // _R),
                                              i % (_NH // _R), 0)),
        pl.BlockSpec((_R, 128), lambda i: (i, 0)),
        _deg_spec,
        _bcast_spec((1, 128)),
        _bcast_spec((1, 128)),
        _bcast_spec((1, 128)),
        _bcast_spec((1, 128)),
        _bcast_spec((1, 128)),
        _bcast_spec((128, 64)),
        _bcast_spec((1, 64)),
        _bcast_spec((1, 64)),
        _bcast_spec((1, 64)),
        _bcast_spec((64, 8)),
        _bcast_spec((1, 8)),
    ],
    out_specs=pl.BlockSpec((_R, 8), lambda i: (i, 0)),
    out_shape=jax.ShapeDtypeStruct((_N, 8), jnp.float32),
)


def kernel(x, edge_index, ln_g, ln_b, W1, b1, bn1_g, bn1_b, bn1_m, bn1_v,
           W2, b2, bn2_g, bn2_b, bn2_m, bn2_v, Wc1, bc1, lnc_g, lnc_b,
           Wc2, bc2):
    src = edge_index[0]
    dst = edge_index[1]
    src_rs = src.reshape(_NS, _NBA, _B)
    src_both = jnp.stack([src_rs, src_rs + _N])
    dst_rs = dst.reshape(_NS, _NBA, _B)
    dst_deg = dst.reshape(_NC, _NS, _NBD, _B)

    r2 = lambda a: a.reshape(1, -1)

    _deg, _agg1, _agg2 = _sc_kernels()
    degp = _deg(dst_deg)
    tab1 = _tc1(x, r2(ln_g), r2(ln_b), W1, degp)
    agg1 = _agg1(tab1.reshape(2 * _N, 128), src_both, dst_rs)
    tab2 = _tc2(agg1.reshape(_NC, _N, 128), tab1, degp, r2(b1), r2(bn1_g),
                r2(bn1_b), r2(bn1_m), r2(bn1_v), W2)
    src_rs2 = src.reshape(_NS, _NB2, _B2)
    dst_rs2 = dst.reshape(_NS, _NB2, _B2)
    agg2 = _agg2(tab2, src_rs2, dst_rs2)
    out = _tc3(agg2.reshape(_NC, _A2ROWS, 128), tab2, degp, r2(b2),
               r2(bn2_g), r2(bn2_b), r2(bn2_m), r2(bn2_v), Wc1, r2(bc1),
               r2(lnc_g), r2(lnc_b), Wc2, r2(bc2))
    return out


# speedup vs baseline: 1.3670x; 1.0018x over previous
import functools

import jax
import jax.numpy as jnp
from jax import lax
from jax.experimental import pallas as pl
from jax.experimental.pallas import tpu as pltpu
from jax.experimental.pallas import tpu_sc as plsc

_N = 10000
_E = 160000
_EPS = 1e-5
_NC = 2
_NS = 16
_NH = _N // _NC
_B = 125
_RPW = _N // _NS
_NBA = _E // _NS // _B
_NP = 10240
_RPQ = _NP // _NS
_NBD = _E // (_NC * _NS) // _B
_B2 = 80
_NB2 = _E // _NS // _B2
_A2TRASH = _NH
_A2ROWS = 5008
_A2RPW = _A2ROWS // _NS
_R = 200


@functools.cache
def _sc_kernels():
    mesh = plsc.VectorSubcoreMesh(core_axis_name="c", subcore_axis_name="s")

    @functools.partial(
        pl.kernel,
        out_type=jax.ShapeDtypeStruct((_NC, _NP, 16), jnp.float32),
        mesh=mesh,
        scratch_types=[
            pltpu.VMEM((_NBD, _B), jnp.int32),
            pltpu.VMEM((_B, 16), jnp.float32),
            pltpu.VMEM((128, 16), jnp.float32),
            pltpu.VMEM_SHARED((_NP, 16), jnp.float32),
            pltpu.SemaphoreType.DMA,
        ],
    )
    def deg(dst_hbm, out_hbm, dstv, ones_v, zeros_v, acc, ssem):
        cid = lax.axis_index("c")
        sid = lax.axis_index("s")

        def fill(r, carry):
            @pl.when(r < _B)
            def _():
                ones_v[r, :] = jnp.full((16,), 1.0, jnp.float32)

            zeros_v[r, :] = jnp.zeros((16,), jnp.float32)
            return carry

        lax.fori_loop(0, 128, fill, 0)
        for i in range(_RPQ // 128):
            pltpu.sync_copy(zeros_v, acc.at[pl.ds(sid * _RPQ + i * 128,
                                                  128)])
        pltpu.sync_copy(dst_hbm.at[cid, sid], dstv)
        plsc.subcore_barrier()

        def body(t, carry):
            da = pltpu.async_copy(ones_v, acc.at[dstv.at[t * 2]], ssem,
                                  add=True)
            db = pltpu.async_copy(ones_v, acc.at[dstv.at[t * 2 + 1]], ssem,
                                  add=True)
            da.wait()
            db.wait()
            return carry

        lax.fori_loop(0, _NBD // 2, body, 0)
        plsc.subcore_barrier()
        pltpu.sync_copy(acc.at[pl.ds(sid * _RPQ, _RPQ)],
                        out_hbm.at[cid, pl.ds(sid * _RPQ, _RPQ)])

    @functools.partial(
        pl.kernel,
        out_type=jax.ShapeDtypeStruct((_NC, _NS, _RPW, 128), jnp.float32),
        mesh=mesh,
        scratch_types=[
            pltpu.VMEM((_NBA, _B), jnp.int32),
            pltpu.VMEM((_NBA, _B), jnp.int32),
            pltpu.VMEM((_B, 128), jnp.float32),
            pltpu.VMEM_SHARED((_N, 128), jnp.float32),
            pltpu.SemaphoreType.DMA,
        ],
    )
    def agg1(tab_hbm, src_hbm, dst_hbm, out_hbm, srcv, dstv, rows0, acc,
             sem0):
        cid = lax.axis_index("c")
        sid = lax.axis_index("s")

        def fillz(r, carry):
            for k in range(8):
                rows0[r, pl.ds(k * 16, 16)] = jnp.zeros((16,), jnp.float32)
            return carry

        lax.fori_loop(0, _B, fillz, 0)
        for i in range(_RPW // _B):
            pltpu.sync_copy(rows0, acc.at[pl.ds(sid * _RPW + i * _B, _B)])
        pltpu.sync_copy(src_hbm.at[cid, sid], srcv)
        pltpu.sync_copy(dst_hbm.at[sid], dstv)
        plsc.subcore_barrier()

        def body(j, carry):
            pltpu.async_copy(tab_hbm.at[srcv.at[j]], rows0, sem0).wait()
            pltpu.sync_copy(rows0, acc.at[dstv.at[j]], add=True)
            return carry

        lax.fori_loop(0, _NBA, body, 0)
        plsc.subcore_barrier()
        pltpu.sync_copy(acc.at[pl.ds(sid * _RPW, _RPW)],
                        out_hbm.at[cid, sid])

    @functools.partial(
        pl.kernel,
        out_type=jax.ShapeDtypeStruct((_NC, _NS, _A2RPW, 128), jnp.float32),
        mesh=mesh,
        scratch_types=[
            pltpu.VMEM((_NB2, _B2), jnp.int32),
            pltpu.VMEM((_NB2, _B2), jnp.int32),
            pltpu.VMEM((_NB2, _B2), jnp.int32),
            pltpu.VMEM((_B2, 128), jnp.float32),
            pltpu.VMEM_SHARED((_A2ROWS, 128), jnp.float32),
            pltpu.SemaphoreType.DMA,
        ],
    )
    def agg2(tab_hbm, src_hbm, dst_hbm, out_hbm, srcv, dstv, dstt, rows0,
             acc, sem0):
        cid = lax.axis_index("c")
        sid = lax.axis_index("s")
        lo = cid * _NH

        def fillz(r, carry):
            for k in range(8):
                rows0[r, pl.ds(k * 16, 16)] = jnp.zeros((16,), jnp.float32)
            return carry

        lax.fori_loop(0, _B2, fillz, 0)
        base = sid * _A2RPW
        for i in range(3):
            pltpu.sync_copy(rows0, acc.at[pl.ds(base + i * _B2, _B2)])
        pltpu.sync_copy(rows0.at[pl.ds(0, _A2RPW - 3 * _B2)],
                        acc.at[pl.ds(base + 3 * _B2, _A2RPW - 3 * _B2)])
        pltpu.sync_copy(src_hbm.at[sid], srcv)
        pltpu.sync_copy(dst_hbm.at[sid], dstv)

        def xform(j, carry):
            for k in range(_B2 // 16):
                v = dstv[j, pl.ds(k * 16, 16)]
                vl = v - lo
                ok = (vl >= 0) & (vl < _NH)
                dstt[j, pl.ds(k * 16, 16)] = jnp.where(ok, vl, _A2TRASH)
            return carry

        lax.fori_loop(0, _NB2, xform, 0)
        plsc.subcore_barrier()

        def body(j, carry):
            pltpu.async_copy(tab_hbm.at[srcv.at[j]], rows0, sem0).wait()
            pltpu.sync_copy(rows0, acc.at[dstt.at[j]], add=True)
            return carry

        lax.fori_loop(0, _NB2, body, 0)
        plsc.subcore_barrier()
        pltpu.sync_copy(acc.at[pl.ds(base, _A2RPW)], out_hbm.at[cid, sid])

    return deg, agg1, agg2


def _dot(a, b):
    return lax.dot_general(a, b, (((1,), (0,)), ((), ())),
                           preferred_element_type=jnp.float32,
                           precision=lax.Precision.HIGHEST)


def _dinv_of(degp):
    deg = degp[0, :, 0:1] + degp[1, :, 0:1] + 1.0
    return lax.rsqrt(deg)


def _tc1_body(x_ref, lng, lnb, w1, degp, out_ref):
    xb = x_ref[...]
    mu = jnp.mean(xb, axis=1, keepdims=True)
    xc = xb - mu
    var = jnp.mean(xc * xc, axis=1, keepdims=True)
    h = xc * lax.rsqrt(var + _EPS) * lng[0] + lnb[0]
    hw = _dot(h, w1[...]) * _dinv_of(degp)
    out_ref[0] = hw[:, 0:128]
    out_ref[1] = hw[:, 128:256]


def _tc2_body(agg, tabp, degp, b1, g1, be1, m1, v1, w2, out_ref):
    dinv = _dinv_of(degp)
    h = jnp.concatenate([agg[0] + tabp[0], agg[1] + tabp[1]], axis=1)
    h = h * dinv + b1[0]
    h = (h - m1[0]) * lax.rsqrt(v1[0] + _EPS) * g1[0] + be1[0]
    h = jnp.maximum(h, 0.0)
    out_ref[...] = _dot(h, w2[...]) * dinv


def _tc3_body(agg, tabp, degp, b2, g2, be2, m2, v2, wc1, bc1, lncg, lncb,
              wc2, bc2, out_ref):
    dinv = _dinv_of(degp)
    h = agg[0] + tabp[...]
    h = h * dinv + b2[0]
    h = (h - m2[0]) * lax.rsqrt(v2[0] + _EPS) * g2[0] + be2[0]
    h = jnp.maximum(h, 0.0)
    hc = _dot(h, wc1[...]) + bc1[0]
    mu = jnp.mean(hc, axis=1, keepdims=True)
    hcc = hc - mu
    var = jnp.mean(hcc * hcc, axis=1, keepdims=True)
    hc = hcc * lax.rsqrt(var + _EPS) * lncg[0] + lncb[0]
    hc = jnp.maximum(hc, 0.0)
    out_ref[...] = _dot(hc, wc2[...]) + bc2[0]


def _bcast_spec(shape):
    return pl.BlockSpec(shape, lambda i: tuple(0 for _ in shape))


_deg_spec = pl.BlockSpec((2, _R, 16), lambda i: (0, i, 0))

_tc1 = pl.pallas_call(
    _tc1_body,
    grid=(_N // _R,),
    in_specs=[
        pl.BlockSpec((_R, 256), lambda i: (i, 0)),
        _bcast_spec((1, 256)),
        _bcast_spec((1, 256)),
        _bcast_spec((256, 256)),
        _deg_spec,
    ],
    out_specs=pl.BlockSpec((2, _R, 128), lambda i: (0, i, 0)),
    out_shape=jax.ShapeDtypeStruct((2, _N, 128), jnp.float32),
)

_tc2 = pl.pallas_call(
    _tc2_body,
    grid=(_N // _R,),
    in_specs=[
        pl.BlockSpec((2, _R, 128), lambda i: (0, i, 0)),
        pl.BlockSpec((2, _R, 128), lambda i: (0, i, 0)),
        _deg_spec,
        _bcast_spec((1, 256)),
        _bcast_spec((1, 256)),
        _bcast_spec((1, 256)),
        _bcast_spec((1, 256)),
        _bcast_spec((1, 256)),
        _bcast_spec((256, 128)),
    ],
    out_specs=pl.BlockSpec((_R, 128), lambda i: (i, 0)),
    out_shape=jax.ShapeDtypeStruct((_N, 128), jnp.float32),
)

_tc3 = pl.pallas_call(
    _tc3_body,
    grid=(_N // _R,),
    in_specs=[
        pl.BlockSpec((1, _R, 128), lambda i: (i // (_NH // _R),
                                              i % (_NH // _R), 0)),
        pl.BlockSpec((_R, 128), lambda i: (i, 0)),
        _deg_spec,
        _bcast_spec((1, 128)),
        _bcast_spec((1, 128)),
        _bcast_spec((1, 128)),
        _bcast_spec((1, 128)),
        _bcast_spec((1, 128)),
        _bcast_spec((128, 64)),
        _bcast_spec((1, 64)),
        _bcast_spec((1, 64)),
        _bcast_spec((1, 64)),
        _bcast_spec((64, 8)),
        _bcast_spec((1, 8)),
    ],
    out_specs=pl.BlockSpec((_R, 8), lambda i: (i, 0)),
    out_shape=jax.ShapeDtypeStruct((_N, 8), jnp.float32),
)


def kernel(x, edge_index, ln_g, ln_b, W1, b1, bn1_g, bn1_b, bn1_m, bn1_v,
           W2, b2, bn2_g, bn2_b, bn2_m, bn2_v, Wc1, bc1, lnc_g, lnc_b,
           Wc2, bc2):
    src = edge_index[0]
    dst = edge_index[1]
    src_rs = src.reshape(_NS, _NBA, _B)
    src_both = jnp.stack([src_rs, src_rs + _N])
    dst_rs = dst.reshape(_NS, _NBA, _B)
    dst_deg = dst.reshape(_NC, _NS, _NBD, _B)

    r2 = lambda a: a.reshape(1, -1)

    _deg, _agg1, _agg2 = _sc_kernels()
    degp = _deg(dst_deg)
    tab1 = _tc1(x, r2(ln_g), r2(ln_b), W1, degp)
    agg1 = _agg1(tab1.reshape(2 * _N, 128), src_both, dst_rs)
    tab2 = _tc2(agg1.reshape(_NC, _N, 128), tab1, degp, r2(b1), r2(bn1_g),
                r2(bn1_b), r2(bn1_m), r2(bn1_v), W2)
    src_rs2 = src.reshape(_NS, _NB2, _B2)
    dst_rs2 = dst.reshape(_NS, _NB2, _B2)
    agg2 = _agg2(tab2, src_rs2, dst_rs2)
    out = _tc3(agg2.reshape(_NC, _A2ROWS, 128), tab2, degp, r2(b2),
               r2(bn2_g), r2(bn2_b), r2(bn2_m), r2(bn2_v), Wc1, r2(bc1),
               r2(lnc_g), r2(lnc_b), Wc2, r2(bc2))
    return out
